# Initial kernel scaffold; baseline (speedup 1.0000x reference)
import jax
import jax.numpy as jnp
from jax.experimental import pallas as pl


def kernel(flat, cu_seqlens, W1, b1, W2, b2, W3, b3, W4, b4, W5, b5):
    raise NotImplementedError("write your pallas kernel here")


# speedup vs baseline: 3.5053x; 3.5053x over previous
import jax
import jax.numpy as jnp
from jax.experimental import pallas as pl
from jax.experimental.pallas import tpu as pltpu

_B = 16
_CHUNK = 2048


def _body(lower_ref, upper_ref, invc_ref, flat_ref,
          w1_ref, b1_ref, w2_ref, b2_ref, w3_ref, b3_ref,
          w4_ref, b4_ref, w5_ref, b5_ref,
          out_ref, acc_ref):
    i = pl.program_id(0)
    x = flat_ref[...]
    h = jnp.maximum(
        jnp.dot(x, w1_ref[...], preferred_element_type=jnp.float32)
        + b1_ref[...], 0.0)
    h = jnp.maximum(
        jnp.dot(h, w2_ref[...], preferred_element_type=jnp.float32)
        + b2_ref[...], 0.0)
    h = jnp.maximum(
        jnp.dot(h, w3_ref[...], preferred_element_type=jnp.float32)
        + b3_ref[...], 0.0)

    row = jax.lax.broadcasted_iota(jnp.int32, (_B, _CHUNK), 1) + i * _CHUNK
    member = (row >= lower_ref[...]) & (row < upper_ref[...])
    onehot = member.astype(jnp.float32) * invc_ref[...]
    part = jnp.dot(onehot, h, preferred_element_type=jnp.float32)

    @pl.when(i == 0)
    def _():
        acc_ref[...] = part

    @pl.when(i > 0)
    def _():
        acc_ref[...] += part

    @pl.when(i == pl.num_programs(0) - 1)
    def _():
        pooled = acc_ref[...]
        z = jnp.maximum(
            jnp.dot(pooled, w4_ref[...], preferred_element_type=jnp.float32)
            + b4_ref[...], 0.0)
        o = jnp.dot(z, w5_ref[...], preferred_element_type=jnp.float32) \
            + b5_ref[...]
        out_ref[...] = jax.nn.sigmoid(o)


def kernel(flat, cu_seqlens, W1, b1, W2, b2, W3, b3, W4, b4, W5, b5):
    total, d_in = flat.shape
    n_chunks = total // _CHUNK

    flat_p = jnp.pad(flat, ((0, 0), (0, 8 - d_in)))
    w1_p = jnp.pad(W1, ((0, 8 - d_in), (0, 0)))

    lower = cu_seqlens[:-1].reshape(_B, 1)
    upper = cu_seqlens[1:].reshape(_B, 1)
    counts = (upper - lower).astype(jnp.float32)
    invc = 1.0 / jnp.maximum(counts, 1.0)

    full = lambda shape: pl.BlockSpec(shape, lambda i: (0, 0))
    in_specs = [
            full((_B, 1)),
            full((_B, 1)),
            full((_B, 1)),
            pl.BlockSpec((_CHUNK, 8), lambda i: (i, 0)),
            full(w1_p.shape), full((1, 64)),
            full(W2.shape), full((1, 128)),
            full(W3.shape), full((1, 256)),
            full(W4.shape), full((1, 512)),
            full(W5.shape), full((1, 2500)),
        ]
    return pl.pallas_call(
        _body,
        grid=(n_chunks,),
        in_specs=in_specs,
        out_specs=full((_B, 2500)),
        out_shape=jax.ShapeDtypeStruct((_B, 2500), jnp.float32),
        scratch_shapes=[pltpu.VMEM((_B, 256), jnp.float32)],
        compiler_params=pltpu.CompilerParams(
            dimension_semantics=("arbitrary",)),
    )(lower, upper, invc, flat_p,
      w1_p, b1.reshape(1, 64), W2, b2.reshape(1, 128),
      W3, b3.reshape(1, 256), W4, b4.reshape(1, 512),
      W5, b5.reshape(1, 2500))


# speedup vs baseline: 4.0075x; 1.1433x over previous
import jax
import jax.numpy as jnp
from jax.experimental import pallas as pl
from jax.experimental.pallas import tpu as pltpu

_B = 16
_CHUNK = 4096


def _body(lower_ref, upper_ref, invc_ref, flat_ref,
          w1_ref, w2_ref, w3_ref, w4_ref, b4_ref, w5_ref, b5_ref,
          out_ref, acc_ref, h1_ref, h2_ref):
    i = pl.program_id(0)

    @pl.when(i == 0)
    def _():
        ones_col1 = (jax.lax.broadcasted_iota(jnp.int32, (_CHUNK, 8), 1)
                     == 0).astype(jnp.float32)
        h1_ref[:, 64:72] = ones_col1
        h2_ref[:, 128:136] = ones_col1

    x = flat_ref[...]
    h1_ref[:, :64] = jnp.maximum(
        jnp.dot(x, w1_ref[...], preferred_element_type=jnp.float32), 0.0)
    h2_ref[:, :128] = jnp.maximum(
        jnp.dot(h1_ref[...], w2_ref[...],
                preferred_element_type=jnp.float32), 0.0)
    h = jnp.maximum(
        jnp.dot(h2_ref[...], w3_ref[...],
                preferred_element_type=jnp.float32), 0.0)

    row = jax.lax.broadcasted_iota(jnp.int32, (_B, _CHUNK), 1) + i * _CHUNK
    member = (row >= lower_ref[...]) & (row < upper_ref[...])
    onehot = member.astype(jnp.float32) * invc_ref[...]
    part = jnp.dot(onehot, h, preferred_element_type=jnp.float32)

    @pl.when(i == 0)
    def _():
        acc_ref[...] = part

    @pl.when(i > 0)
    def _():
        acc_ref[...] += part

    @pl.when(i == pl.num_programs(0) - 1)
    def _():
        pooled = acc_ref[...]
        z = jnp.maximum(
            jnp.dot(pooled, w4_ref[...], preferred_element_type=jnp.float32)
            + b4_ref[...], 0.0)
        o = jnp.dot(z, w5_ref[...], preferred_element_type=jnp.float32) \
            + b5_ref[...]
        out_ref[...] = jax.nn.sigmoid(o)


def kernel(flat, cu_seqlens, W1, b1, W2, b2, W3, b3, W4, b4, W5, b5):
    total, d_in = flat.shape
    n_chunks = total // _CHUNK

    ones = jnp.ones((total, 1), jnp.float32)
    flat_a = jnp.concatenate(
        [flat, ones, jnp.zeros((total, 4), jnp.float32)], axis=1)
    w1_a = jnp.concatenate(
        [W1, b1[None, :], jnp.zeros((4, 64), jnp.float32)], axis=0)
    w2_a = jnp.concatenate(
        [W2, b2[None, :], jnp.zeros((7, 128), jnp.float32)], axis=0)
    w3_a = jnp.concatenate(
        [W3, b3[None, :], jnp.zeros((7, 256), jnp.float32)], axis=0)

    lower = cu_seqlens[:-1].reshape(_B, 1)
    upper = cu_seqlens[1:].reshape(_B, 1)
    counts = (upper - lower).astype(jnp.float32)
    invc = 1.0 / jnp.maximum(counts, 1.0)

    full = lambda shape: pl.BlockSpec(shape, lambda i: (0, 0))
    in_specs = [
        full((_B, 1)),
        full((_B, 1)),
        full((_B, 1)),
        pl.BlockSpec((_CHUNK, 8), lambda i: (i, 0)),
        full(w1_a.shape), full(w2_a.shape), full(w3_a.shape),
        full(W4.shape), full((1, 512)), full(W5.shape), full((1, 2500)),
    ]
    return pl.pallas_call(
        _body,
        grid=(n_chunks,),
        in_specs=in_specs,
        out_specs=full((_B, 2500)),
        out_shape=jax.ShapeDtypeStruct((_B, 2500), jnp.float32),
        scratch_shapes=[
            pltpu.VMEM((_B, 256), jnp.float32),
            pltpu.VMEM((_CHUNK, 72), jnp.float32),
            pltpu.VMEM((_CHUNK, 136), jnp.float32),
        ],
        compiler_params=pltpu.CompilerParams(
            dimension_semantics=("arbitrary",)),
    )(lower, upper, invc, flat_a, w1_a, w2_a, w3_a,
      W4, b4.reshape(1, 512), W5, b5.reshape(1, 2500))
